# Initial kernel scaffold; baseline (speedup 1.0000x reference)
import jax
import jax.numpy as jnp
from jax.experimental import pallas as pl


def kernel(x, edge_index, W1, b1, W2, b2):
    raise NotImplementedError("write your pallas kernel here")


# speedup vs baseline: 26.1000x; 26.1000x over previous
import functools

import jax
import jax.numpy as jnp
from jax import lax
from jax.experimental import pallas as pl
from jax.experimental.pallas import tpu as pltpu
from jax.experimental.pallas import tpu_sc as plsc

N_NODES = 10000
N_EDGES = 320000
D_IN = 128
D_HID = 16
D_OUT = 12

NC, NS = 2, 16
NW = NC * NS
CHUNK = 128
NCH = 80
EPW = NCH * CHUNK
EPAD = EPW * NW
NPAD = 10240
RPS = NPAD // NS
F = 16
BLK = 512
NBLK = NPAD // BLK

_MESH = plsc.VectorSubcoreMesh(core_axis_name="c", subcore_axis_name="s")
_SC_PARAMS = pltpu.CompilerParams(use_tc_tiling_on_sc=False)


@functools.partial(
    pl.kernel,
    out_type=jax.ShapeDtypeStruct((NC, NPAD, F), jnp.float32),
    mesh=_MESH,
    scratch_types=[
        pltpu.VMEM((RPS, F), jnp.float32),
        pltpu.VMEM((CHUNK, F), jnp.float32),
        pltpu.VMEM((NCH, CHUNK), jnp.int32),
        pltpu.VMEM_SHARED((NPAD, F), jnp.float32),
    ],
    compiler_params=_SC_PARAMS,
)
def _sc_degree(dst_hbm, out_hbm, zbuf, ones, idx, deg_sp):
    cid = lax.axis_index("c")
    sid = lax.axis_index("s")
    w = cid * NS + sid

    def fill_z(i, c):
        zbuf[i] = jnp.zeros((F,), jnp.float32)
        return c
    lax.fori_loop(0, RPS, fill_z, 0)

    def fill_o(i, c):
        ones[i] = jnp.ones((F,), jnp.float32)
        return c
    lax.fori_loop(0, CHUNK, fill_o, 0)

    pltpu.sync_copy(zbuf, deg_sp.at[pl.ds(sid * RPS, RPS)])
    pltpu.sync_copy(dst_hbm.at[w], idx)
    plsc.subcore_barrier()

    def body(j, c):
        pltpu.sync_copy(ones, deg_sp.at[idx.at[j]], add=True)
        return c
    lax.fori_loop(0, NCH, body, 0)

    plsc.subcore_barrier()
    pltpu.sync_copy(deg_sp.at[pl.ds(sid * RPS, RPS)],
                    out_hbm.at[cid, pl.ds(sid * RPS, RPS)])


@functools.partial(
    pl.kernel,
    out_type=jax.ShapeDtypeStruct((NC, NPAD, F), jnp.float32),
    mesh=_MESH,
    scratch_types=[
        pltpu.VMEM((RPS, F), jnp.float32),
        pltpu.VMEM((NCH, CHUNK), jnp.int32),
        pltpu.VMEM((NCH, CHUNK), jnp.int32),
        pltpu.VMEM((CHUNK, F), jnp.float32),
        pltpu.VMEM_SHARED((NPAD, F), jnp.float32),
    ],
    compiler_params=_SC_PARAMS,
)
def _sc_agg(g_hbm, src_hbm, dst_hbm, out_hbm, zbuf, sidx, didx, rows, agg_sp):
    cid = lax.axis_index("c")
    sid = lax.axis_index("s")
    w = cid * NS + sid

    def fill_z(i, c):
        zbuf[i] = jnp.zeros((F,), jnp.float32)
        return c
    lax.fori_loop(0, RPS, fill_z, 0)

    pltpu.sync_copy(zbuf, agg_sp.at[pl.ds(sid * RPS, RPS)])
    pltpu.sync_copy(src_hbm.at[w], sidx)
    pltpu.sync_copy(dst_hbm.at[w], didx)
    plsc.subcore_barrier()

    def body(j, c):
        pltpu.sync_copy(g_hbm.at[sidx.at[j]], rows)
        pltpu.sync_copy(rows, agg_sp.at[didx.at[j]], add=True)
        return c
    lax.fori_loop(0, NCH, body, 0)

    plsc.subcore_barrier()
    pltpu.sync_copy(agg_sp.at[pl.ds(sid * RPS, RPS)],
                    out_hbm.at[cid, pl.ds(sid * RPS, RPS)])


def _pre_body(x_ref, w1_ref, degp_ref, g1_ref, dis_ref):
    deg = degp_ref[0] + degp_ref[1] + 1.0
    dis = lax.rsqrt(deg)
    h = jnp.dot(x_ref[...], w1_ref[...], preferred_element_type=jnp.float32)
    g1_ref[...] = dis * h
    dis_ref[...] = dis


def _mid_body(g1_ref, aggp_ref, dis_ref, b1_ref, w2_ref, g2_ref):
    dis = dis_ref[...]
    o = dis * (aggp_ref[0] + aggp_ref[1] + g1_ref[...]) + b1_ref[...]
    o = jnp.maximum(o, 0.0)
    h2 = jnp.dot(o, w2_ref[...], preferred_element_type=jnp.float32)
    g2_ref[...] = dis * h2


def _post_body(g2_ref, aggp_ref, dis_ref, b2_ref, out_ref):
    o = dis_ref[...] * (aggp_ref[0] + aggp_ref[1] + g2_ref[...]) + b2_ref[...]
    col = lax.broadcasted_iota(jnp.int32, (BLK, F), 1)
    valid = col < D_OUT
    mx = jnp.max(jnp.where(valid, o, -jnp.inf), axis=1, keepdims=True)
    ex = jnp.where(valid, jnp.exp(o - mx), 0.0)
    lse = jnp.log(jnp.sum(ex, axis=1, keepdims=True))
    out_ref[...] = o - mx - lse


_node_spec = pl.BlockSpec((BLK, F), lambda i: (i, 0))
_part_spec = pl.BlockSpec((NC, BLK, F), lambda i: (0, i, 0))

_pre_call = pl.pallas_call(
    _pre_body,
    grid=(NBLK,),
    in_specs=[
        pl.BlockSpec((BLK, D_IN), lambda i: (i, 0)),
        pl.BlockSpec((D_IN, D_HID), lambda i: (0, 0)),
        _part_spec,
    ],
    out_specs=[_node_spec, _node_spec],
    out_shape=[
        jax.ShapeDtypeStruct((NPAD, F), jnp.float32),
        jax.ShapeDtypeStruct((NPAD, F), jnp.float32),
    ],
)

_mid_call = pl.pallas_call(
    _mid_body,
    grid=(NBLK,),
    in_specs=[
        _node_spec,
        _part_spec,
        _node_spec,
        pl.BlockSpec((1, F), lambda i: (0, 0)),
        pl.BlockSpec((F, F), lambda i: (0, 0)),
    ],
    out_specs=_node_spec,
    out_shape=jax.ShapeDtypeStruct((NPAD, F), jnp.float32),
)

_post_call = pl.pallas_call(
    _post_body,
    grid=(NBLK,),
    in_specs=[
        _node_spec,
        _part_spec,
        _node_spec,
        pl.BlockSpec((1, F), lambda i: (0, 0)),
    ],
    out_specs=_node_spec,
    out_shape=jax.ShapeDtypeStruct((NPAD, F), jnp.float32),
)


def kernel(x, edge_index, W1, b1, W2, b2):
    ei = edge_index.astype(jnp.int32)
    pad = jnp.full((EPAD - N_EDGES,), N_NODES, jnp.int32)
    src_r = jnp.concatenate([ei[0], pad]).reshape(NW, NCH, CHUNK)
    dst_r = jnp.concatenate([ei[1], pad]).reshape(NW, NCH, CHUNK)
    xp = jnp.concatenate(
        [x, jnp.zeros((NPAD - N_NODES, D_IN), jnp.float32)], axis=0)
    w2p = jnp.concatenate(
        [W2, jnp.zeros((D_HID, F - D_OUT), jnp.float32)], axis=1)
    b1r = b1.reshape(1, D_HID)
    b2r = jnp.concatenate(
        [b2, jnp.zeros((F - D_OUT,), jnp.float32)]).reshape(1, F)

    degp = _sc_degree(dst_r)
    g1, dis = _pre_call(xp, W1, degp)
    agg1 = _sc_agg(g1, src_r, dst_r)
    g2 = _mid_call(g1, agg1, dis, b1r, w2p)
    agg2 = _sc_agg(g2, src_r, dst_r)
    outp = _post_call(g2, agg2, dis, b2r)
    return outp[:N_NODES, :D_OUT]


# speedup vs baseline: 31.4238x; 1.2040x over previous
import functools

import jax
import jax.numpy as jnp
from jax import lax
from jax.experimental import pallas as pl
from jax.experimental.pallas import tpu as pltpu
from jax.experimental.pallas import tpu_sc as plsc

N_NODES = 10000
N_EDGES = 320000
D_IN = 128
D_HID = 16
D_OUT = 12

NC, NS = 2, 16
NW = NC * NS
CHUNK = 128
NCH = 80
EPW = NCH * CHUNK
EPAD = EPW * NW
NPAD = 10240
RPS = NPAD // NS
F = 16
BLK = 512
NBLK = NPAD // BLK

_MESH = plsc.VectorSubcoreMesh(core_axis_name="c", subcore_axis_name="s")
_SC_PARAMS = pltpu.CompilerParams(use_tc_tiling_on_sc=False)


@functools.partial(
    pl.kernel,
    out_type=jax.ShapeDtypeStruct((NC, NPAD, F), jnp.float32),
    mesh=_MESH,
    scratch_types=[
        pltpu.VMEM((RPS, F), jnp.float32),
        pltpu.VMEM((CHUNK, F), jnp.float32),
        pltpu.VMEM((NCH, CHUNK), jnp.int32),
        pltpu.VMEM_SHARED((NPAD, F), jnp.float32),
        pltpu.SemaphoreType.DMA,
    ],
    compiler_params=_SC_PARAMS,
)
def _sc_degree(dst_hbm, out_hbm, zbuf, ones, idx, deg_sp, ssem):
    cid = lax.axis_index("c")
    sid = lax.axis_index("s")
    w = cid * NS + sid

    def fill_z(i, c):
        zbuf[i] = jnp.zeros((F,), jnp.float32)
        return c
    lax.fori_loop(0, RPS, fill_z, 0)

    def fill_o(i, c):
        ones[i] = jnp.ones((F,), jnp.float32)
        return c
    lax.fori_loop(0, CHUNK, fill_o, 0)

    pltpu.sync_copy(zbuf, deg_sp.at[pl.ds(sid * RPS, RPS)])
    pltpu.sync_copy(dst_hbm.at[w], idx)
    plsc.subcore_barrier()

    def body(t, c):
        base = t * 16
        ds = [pltpu.async_copy(ones, deg_sp.at[idx.at[base + b]], ssem,
                               add=True)
              for b in range(16)]
        for d in ds:
            d.wait()
        return c
    lax.fori_loop(0, NCH // 16, body, 0)

    plsc.subcore_barrier()
    pltpu.sync_copy(deg_sp.at[pl.ds(sid * RPS, RPS)],
                    out_hbm.at[cid, pl.ds(sid * RPS, RPS)])


@functools.partial(
    pl.kernel,
    out_type=jax.ShapeDtypeStruct((NC, NPAD, F), jnp.float32),
    mesh=_MESH,
    scratch_types=[
        pltpu.VMEM((RPS, F), jnp.float32),
        pltpu.VMEM((NCH, CHUNK), jnp.int32),
        pltpu.VMEM((NCH, CHUNK), jnp.int32),
        pltpu.VMEM((8, CHUNK, F), jnp.float32),
        pltpu.VMEM_SHARED((NPAD, F), jnp.float32),
        pltpu.SemaphoreType.DMA,
        pltpu.SemaphoreType.DMA,
    ],
    compiler_params=_SC_PARAMS,
)
def _sc_agg(g_hbm, src_hbm, dst_hbm, out_hbm, zbuf, sidx, didx, rows, agg_sp,
            gsem, ssem):
    cid = lax.axis_index("c")
    sid = lax.axis_index("s")
    w = cid * NS + sid

    def fill_z(i, c):
        zbuf[i] = jnp.zeros((F,), jnp.float32)
        return c
    lax.fori_loop(0, RPS, fill_z, 0)

    pltpu.sync_copy(zbuf, agg_sp.at[pl.ds(sid * RPS, RPS)])
    pltpu.sync_copy(src_hbm.at[w], sidx)
    pltpu.sync_copy(dst_hbm.at[w], didx)
    plsc.subcore_barrier()

    def body(t, c):
        base = t * 8
        gds = [pltpu.async_copy(g_hbm.at[sidx.at[base + b]], rows.at[b], gsem)
               for b in range(8)]
        for d in gds:
            d.wait()
        sds = [pltpu.async_copy(rows.at[b], agg_sp.at[didx.at[base + b]],
                                ssem, add=True)
               for b in range(8)]
        for d in sds:
            d.wait()
        return c
    lax.fori_loop(0, NCH // 8, body, 0)

    plsc.subcore_barrier()
    pltpu.sync_copy(agg_sp.at[pl.ds(sid * RPS, RPS)],
                    out_hbm.at[cid, pl.ds(sid * RPS, RPS)])


def _pre_body(x_ref, w1_ref, degp_ref, g1_ref, dis_ref):
    deg = degp_ref[0] + degp_ref[1] + 1.0
    dis = lax.rsqrt(deg)
    h = jnp.dot(x_ref[...], w1_ref[...], preferred_element_type=jnp.float32)
    g1_ref[...] = dis * h
    dis_ref[...] = dis


def _mid_body(g1_ref, aggp_ref, dis_ref, b1_ref, w2_ref, g2_ref):
    dis = dis_ref[...]
    o = dis * (aggp_ref[0] + aggp_ref[1] + g1_ref[...]) + b1_ref[...]
    o = jnp.maximum(o, 0.0)
    h2 = jnp.dot(o, w2_ref[...], preferred_element_type=jnp.float32)
    g2_ref[...] = dis * h2


def _post_body(g2_ref, aggp_ref, dis_ref, b2_ref, out_ref):
    o = dis_ref[...] * (aggp_ref[0] + aggp_ref[1] + g2_ref[...]) + b2_ref[...]
    col = lax.broadcasted_iota(jnp.int32, (BLK, F), 1)
    valid = col < D_OUT
    mx = jnp.max(jnp.where(valid, o, -jnp.inf), axis=1, keepdims=True)
    ex = jnp.where(valid, jnp.exp(o - mx), 0.0)
    lse = jnp.log(jnp.sum(ex, axis=1, keepdims=True))
    out_ref[...] = o - mx - lse


_node_spec = pl.BlockSpec((BLK, F), lambda i: (i, 0))
_part_spec = pl.BlockSpec((NC, BLK, F), lambda i: (0, i, 0))

_pre_call = pl.pallas_call(
    _pre_body,
    grid=(NBLK,),
    in_specs=[
        pl.BlockSpec((BLK, D_IN), lambda i: (i, 0)),
        pl.BlockSpec((D_IN, D_HID), lambda i: (0, 0)),
        _part_spec,
    ],
    out_specs=[_node_spec, _node_spec],
    out_shape=[
        jax.ShapeDtypeStruct((NPAD, F), jnp.float32),
        jax.ShapeDtypeStruct((NPAD, F), jnp.float32),
    ],
)

_mid_call = pl.pallas_call(
    _mid_body,
    grid=(NBLK,),
    in_specs=[
        _node_spec,
        _part_spec,
        _node_spec,
        pl.BlockSpec((1, F), lambda i: (0, 0)),
        pl.BlockSpec((F, F), lambda i: (0, 0)),
    ],
    out_specs=_node_spec,
    out_shape=jax.ShapeDtypeStruct((NPAD, F), jnp.float32),
)

_post_call = pl.pallas_call(
    _post_body,
    grid=(NBLK,),
    in_specs=[
        _node_spec,
        _part_spec,
        _node_spec,
        pl.BlockSpec((1, F), lambda i: (0, 0)),
    ],
    out_specs=_node_spec,
    out_shape=jax.ShapeDtypeStruct((NPAD, F), jnp.float32),
)


def kernel(x, edge_index, W1, b1, W2, b2):
    ei = edge_index.astype(jnp.int32)
    pad = jnp.full((EPAD - N_EDGES,), N_NODES, jnp.int32)
    src_r = jnp.concatenate([ei[0], pad]).reshape(NW, NCH, CHUNK)
    dst_r = jnp.concatenate([ei[1], pad]).reshape(NW, NCH, CHUNK)
    xp = jnp.concatenate(
        [x, jnp.zeros((NPAD - N_NODES, D_IN), jnp.float32)], axis=0)
    w2p = jnp.concatenate(
        [W2, jnp.zeros((D_HID, F - D_OUT), jnp.float32)], axis=1)
    b1r = b1.reshape(1, D_HID)
    b2r = jnp.concatenate(
        [b2, jnp.zeros((F - D_OUT,), jnp.float32)]).reshape(1, F)

    degp = _sc_degree(dst_r)
    g1, dis = _pre_call(xp, W1, degp)
    agg1 = _sc_agg(g1, src_r, dst_r)
    g2 = _mid_call(g1, agg1, dis, b1r, w2p)
    agg2 = _sc_agg(g2, src_r, dst_r)
    outp = _post_call(g2, agg2, dis, b2r)
    return outp[:N_NODES, :D_OUT]


# speedup vs baseline: 45.0814x; 1.4346x over previous
import functools

import jax
import jax.numpy as jnp
from jax import lax
from jax.experimental import pallas as pl
from jax.experimental.pallas import tpu as pltpu
from jax.experimental.pallas import tpu_sc as plsc

N_NODES = 10000
N_EDGES = 320000
D_IN = 128
D_HID = 16
D_OUT = 12

NC, NS = 2, 16
NW = NC * NS
CHUNK = 128
NCH = 80
EPW = NCH * CHUNK
EPAD = EPW * NW
NPAD = 10240
RPS = NPAD // NS
F = 16
BLK = 512
NBLK = NPAD // BLK

_MESH = plsc.VectorSubcoreMesh(core_axis_name="c", subcore_axis_name="s")
_SC_PARAMS = pltpu.CompilerParams(use_tc_tiling_on_sc=False)


@functools.partial(
    pl.kernel,
    out_type=jax.ShapeDtypeStruct((NC, NPAD, F), jnp.float32),
    mesh=_MESH,
    scratch_types=[
        pltpu.VMEM((RPS, F), jnp.float32),
        pltpu.VMEM((CHUNK, F), jnp.float32),
        pltpu.VMEM((NCH, CHUNK), jnp.int32),
        pltpu.VMEM_SHARED((NPAD, F), jnp.float32),
        pltpu.SemaphoreType.DMA,
    ],
    compiler_params=_SC_PARAMS,
)
def _sc_degree(dst_hbm, out_hbm, zbuf, ones, idx, deg_sp, ssem):
    cid = lax.axis_index("c")
    sid = lax.axis_index("s")
    w = cid * NS + sid

    def fill_z(i, c):
        zbuf[i] = jnp.zeros((F,), jnp.float32)
        return c
    lax.fori_loop(0, RPS, fill_z, 0)

    def fill_o(i, c):
        ones[i] = jnp.ones((F,), jnp.float32)
        return c
    lax.fori_loop(0, CHUNK, fill_o, 0)

    pltpu.sync_copy(zbuf, deg_sp.at[pl.ds(sid * RPS, RPS)])
    pltpu.sync_copy(dst_hbm.at[w], idx)
    plsc.subcore_barrier()

    def body(t, c):
        base = t * 16
        ds = [pltpu.async_copy(ones, deg_sp.at[idx.at[base + b]], ssem,
                               add=True)
              for b in range(16)]
        for d in ds:
            d.wait()
        return c
    lax.fori_loop(0, NCH // 16, body, 0)

    plsc.subcore_barrier()
    pltpu.sync_copy(deg_sp.at[pl.ds(sid * RPS, RPS)],
                    out_hbm.at[cid, pl.ds(sid * RPS, RPS)])


@functools.partial(
    pl.kernel,
    out_type=jax.ShapeDtypeStruct((NC, NPAD, F), jnp.float32),
    mesh=_MESH,
    scratch_types=[
        pltpu.VMEM((RPS, F), jnp.float32),
        pltpu.VMEM((NCH, CHUNK), jnp.int32),
        pltpu.VMEM((NCH, CHUNK), jnp.int32),
        pltpu.VMEM((8, CHUNK, F), jnp.float32),
        pltpu.VMEM_SHARED((NPAD, F), jnp.float32),
        pltpu.VMEM_SHARED((NPAD, F), jnp.float32),
        pltpu.SemaphoreType.DMA,
        pltpu.SemaphoreType.DMA,
    ],
    compiler_params=_SC_PARAMS,
)
def _sc_agg(g_hbm, src_hbm, dst_hbm, out_hbm, zbuf, sidx, didx, rows, agg_sp,
            g_sp, gsem, ssem):
    cid = lax.axis_index("c")
    sid = lax.axis_index("s")
    w = cid * NS + sid

    def fill_z(i, c):
        zbuf[i] = jnp.zeros((F,), jnp.float32)
        return c
    lax.fori_loop(0, RPS, fill_z, 0)

    pltpu.sync_copy(zbuf, agg_sp.at[pl.ds(sid * RPS, RPS)])
    pltpu.sync_copy(g_hbm.at[pl.ds(sid * RPS, RPS)],
                    g_sp.at[pl.ds(sid * RPS, RPS)])
    pltpu.sync_copy(src_hbm.at[w], sidx)
    pltpu.sync_copy(dst_hbm.at[w], didx)
    plsc.subcore_barrier()

    def body(t, c):
        base = t * 8
        gds = [pltpu.async_copy(g_sp.at[sidx.at[base + b]], rows.at[b], gsem)
               for b in range(8)]
        for d in gds:
            d.wait()
        sds = [pltpu.async_copy(rows.at[b], agg_sp.at[didx.at[base + b]],
                                ssem, add=True)
               for b in range(8)]
        for d in sds:
            d.wait()
        return c
    lax.fori_loop(0, NCH // 8, body, 0)

    plsc.subcore_barrier()
    pltpu.sync_copy(agg_sp.at[pl.ds(sid * RPS, RPS)],
                    out_hbm.at[cid, pl.ds(sid * RPS, RPS)])


def _pre_body(x_ref, w1_ref, degp_ref, g1_ref, dis_ref):
    deg = degp_ref[0] + degp_ref[1] + 1.0
    dis = lax.rsqrt(deg)
    h = jnp.dot(x_ref[...], w1_ref[...], preferred_element_type=jnp.float32)
    g1_ref[...] = dis * h
    dis_ref[...] = dis


def _mid_body(g1_ref, aggp_ref, dis_ref, b1_ref, w2_ref, g2_ref):
    dis = dis_ref[...]
    o = dis * (aggp_ref[0] + aggp_ref[1] + g1_ref[...]) + b1_ref[...]
    o = jnp.maximum(o, 0.0)
    h2 = jnp.dot(o, w2_ref[...], preferred_element_type=jnp.float32)
    g2_ref[...] = dis * h2


def _post_body(g2_ref, aggp_ref, dis_ref, b2_ref, out_ref):
    o = dis_ref[...] * (aggp_ref[0] + aggp_ref[1] + g2_ref[...]) + b2_ref[...]
    col = lax.broadcasted_iota(jnp.int32, (BLK, F), 1)
    valid = col < D_OUT
    mx = jnp.max(jnp.where(valid, o, -jnp.inf), axis=1, keepdims=True)
    ex = jnp.where(valid, jnp.exp(o - mx), 0.0)
    lse = jnp.log(jnp.sum(ex, axis=1, keepdims=True))
    out_ref[...] = o - mx - lse


_node_spec = pl.BlockSpec((BLK, F), lambda i: (i, 0))
_part_spec = pl.BlockSpec((NC, BLK, F), lambda i: (0, i, 0))

_pre_call = pl.pallas_call(
    _pre_body,
    grid=(NBLK,),
    in_specs=[
        pl.BlockSpec((BLK, D_IN), lambda i: (i, 0)),
        pl.BlockSpec((D_IN, D_HID), lambda i: (0, 0)),
        _part_spec,
    ],
    out_specs=[_node_spec, _node_spec],
    out_shape=[
        jax.ShapeDtypeStruct((NPAD, F), jnp.float32),
        jax.ShapeDtypeStruct((NPAD, F), jnp.float32),
    ],
)

_mid_call = pl.pallas_call(
    _mid_body,
    grid=(NBLK,),
    in_specs=[
        _node_spec,
        _part_spec,
        _node_spec,
        pl.BlockSpec((1, F), lambda i: (0, 0)),
        pl.BlockSpec((F, F), lambda i: (0, 0)),
    ],
    out_specs=_node_spec,
    out_shape=jax.ShapeDtypeStruct((NPAD, F), jnp.float32),
)

_post_call = pl.pallas_call(
    _post_body,
    grid=(NBLK,),
    in_specs=[
        _node_spec,
        _part_spec,
        _node_spec,
        pl.BlockSpec((1, F), lambda i: (0, 0)),
    ],
    out_specs=_node_spec,
    out_shape=jax.ShapeDtypeStruct((NPAD, F), jnp.float32),
)


def kernel(x, edge_index, W1, b1, W2, b2):
    ei = edge_index.astype(jnp.int32)
    pad = jnp.full((EPAD - N_EDGES,), N_NODES, jnp.int32)
    src_r = jnp.concatenate([ei[0], pad]).reshape(NW, NCH, CHUNK)
    dst_r = jnp.concatenate([ei[1], pad]).reshape(NW, NCH, CHUNK)
    xp = jnp.concatenate(
        [x, jnp.zeros((NPAD - N_NODES, D_IN), jnp.float32)], axis=0)
    w2p = jnp.concatenate(
        [W2, jnp.zeros((D_HID, F - D_OUT), jnp.float32)], axis=1)
    b1r = b1.reshape(1, D_HID)
    b2r = jnp.concatenate(
        [b2, jnp.zeros((F - D_OUT,), jnp.float32)]).reshape(1, F)

    degp = _sc_degree(dst_r)
    g1, dis = _pre_call(xp, W1, degp)
    agg1 = _sc_agg(g1, src_r, dst_r)
    g2 = _mid_call(g1, agg1, dis, b1r, w2p)
    agg2 = _sc_agg(g2, src_r, dst_r)
    outp = _post_call(g2, agg2, dis, b2r)
    return outp[:N_NODES, :D_OUT]


# speedup vs baseline: 50.3220x; 1.1162x over previous
import functools

import jax
import jax.numpy as jnp
from jax import lax
from jax.experimental import pallas as pl
from jax.experimental.pallas import tpu as pltpu
from jax.experimental.pallas import tpu_sc as plsc

N_NODES = 10000
N_EDGES = 320000
D_IN = 128
D_HID = 16
D_OUT = 12

NC, NS = 2, 16
NW = NC * NS
CHUNK = 128
NCH = 80
EPW = NCH * CHUNK
EPAD = EPW * NW
NPAD = 10240
RPS = NPAD // NS
F = 16
BLK = 512
NBLK = NPAD // BLK

_MESH = plsc.VectorSubcoreMesh(core_axis_name="c", subcore_axis_name="s")
_SC_PARAMS = pltpu.CompilerParams(use_tc_tiling_on_sc=False)


@functools.partial(
    pl.kernel,
    out_type=jax.ShapeDtypeStruct((NC, NPAD, F), jnp.float32),
    mesh=_MESH,
    scratch_types=[
        pltpu.VMEM((RPS, F), jnp.float32),
        pltpu.VMEM((CHUNK, F), jnp.float32),
        pltpu.VMEM((NCH, CHUNK), jnp.int32),
        pltpu.VMEM_SHARED((NPAD, F), jnp.float32),
        pltpu.SemaphoreType.DMA,
    ],
    compiler_params=_SC_PARAMS,
)
def _sc_degree(dst_hbm, out_hbm, zbuf, ones, idx, deg_sp, ssem):
    cid = lax.axis_index("c")
    sid = lax.axis_index("s")
    w = cid * NS + sid

    def fill_z(i, c):
        zbuf[i] = jnp.zeros((F,), jnp.float32)
        return c
    lax.fori_loop(0, RPS, fill_z, 0)

    def fill_o(i, c):
        ones[i] = jnp.ones((F,), jnp.float32)
        return c
    lax.fori_loop(0, CHUNK, fill_o, 0)

    pltpu.sync_copy(zbuf, deg_sp.at[pl.ds(sid * RPS, RPS)])
    pltpu.sync_copy(dst_hbm.at[w], idx)
    plsc.subcore_barrier()

    def body(t, c):
        base = t * 16
        ds = [pltpu.async_copy(ones, deg_sp.at[idx.at[base + b]], ssem,
                               add=True)
              for b in range(16)]
        for d in ds:
            d.wait()
        return c
    lax.fori_loop(0, NCH // 16, body, 0)

    plsc.subcore_barrier()
    pltpu.sync_copy(deg_sp.at[pl.ds(sid * RPS, RPS)],
                    out_hbm.at[cid, pl.ds(sid * RPS, RPS)])


@functools.partial(
    pl.kernel,
    out_type=jax.ShapeDtypeStruct((NC, NPAD, F), jnp.float32),
    mesh=_MESH,
    scratch_types=[
        pltpu.VMEM((RPS, F), jnp.float32),
        pltpu.VMEM((NCH, CHUNK), jnp.int32),
        pltpu.VMEM((NCH, CHUNK), jnp.int32),
        pltpu.VMEM((8, CHUNK, F), jnp.float32),
        pltpu.VMEM_SHARED((NPAD, F), jnp.float32),
        pltpu.VMEM_SHARED((NPAD, F), jnp.float32),
        pltpu.SemaphoreType.DMA,
        pltpu.SemaphoreType.DMA,
    ],
    compiler_params=_SC_PARAMS,
)
def _sc_agg(g_hbm, src_hbm, dst_hbm, out_hbm, zbuf, sidx, didx, rows, agg_sp,
            g_sp, gsem, ssem):
    cid = lax.axis_index("c")
    sid = lax.axis_index("s")
    w = cid * NS + sid

    def fill_z(i, c):
        zbuf[i] = jnp.zeros((F,), jnp.float32)
        return c
    lax.fori_loop(0, RPS, fill_z, 0)

    pltpu.sync_copy(zbuf, agg_sp.at[pl.ds(sid * RPS, RPS)])
    pltpu.sync_copy(g_hbm.at[pl.ds(sid * RPS, RPS)],
                    g_sp.at[pl.ds(sid * RPS, RPS)])
    pltpu.sync_copy(src_hbm.at[w], sidx)
    pltpu.sync_copy(dst_hbm.at[w], didx)
    plsc.subcore_barrier()

    def body(t, c):
        base = t * 8
        gds = [pltpu.async_copy(g_sp.at[sidx.at[base + b]], rows.at[b], gsem)
               for b in range(8)]
        for d in gds:
            d.wait()
        sds = [pltpu.async_copy(rows.at[b], agg_sp.at[didx.at[base + b]],
                                ssem, add=True)
               for b in range(8)]
        for d in sds:
            d.wait()
        return c
    lax.fori_loop(0, NCH // 8, body, 0)

    plsc.subcore_barrier()
    pltpu.sync_copy(agg_sp.at[pl.ds(sid * RPS, RPS)],
                    out_hbm.at[cid, pl.ds(sid * RPS, RPS)])


def _pre_body(x_ref, w1_ref, degp_ref, g1_ref, dis_ref):
    deg = degp_ref[0] + degp_ref[1] + 1.0
    dis = lax.rsqrt(deg)
    h = jnp.dot(x_ref[...], w1_ref[...], preferred_element_type=jnp.float32)
    g1_ref[...] = dis * h
    dis_ref[...] = dis


def _mid_body(g1_ref, aggp_ref, dis_ref, b1_ref, w2_ref, g2_ref):
    dis = dis_ref[...]
    o = dis * (aggp_ref[0] + aggp_ref[1] + g1_ref[...]) + b1_ref[...]
    o = jnp.maximum(o, 0.0)
    h2 = jnp.dot(o, w2_ref[...], preferred_element_type=jnp.float32)
    g2_ref[...] = dis * h2


def _post_body(g2_ref, aggp_ref, dis_ref, b2_ref, out_ref):
    o = dis_ref[...] * (aggp_ref[0] + aggp_ref[1] + g2_ref[...]) + b2_ref[...]
    col = lax.broadcasted_iota(jnp.int32, (NPAD, F), 1)
    valid = col < D_OUT
    mx = jnp.max(jnp.where(valid, o, -jnp.inf), axis=1, keepdims=True)
    ex = jnp.where(valid, jnp.exp(o - mx), 0.0)
    lse = jnp.log(jnp.sum(ex, axis=1, keepdims=True))
    out_ref[...] = (o - mx - lse)[:N_NODES, :D_OUT]


_node_spec = pl.BlockSpec((NPAD, F), lambda: (0, 0))
_part_spec = pl.BlockSpec((NC, NPAD, F), lambda: (0, 0, 0))

_pre_call = pl.pallas_call(
    _pre_body,
    grid=(),
    in_specs=[
        pl.BlockSpec((NPAD, D_IN), lambda: (0, 0)),
        pl.BlockSpec((D_IN, D_HID), lambda: (0, 0)),
        _part_spec,
    ],
    out_specs=[_node_spec, _node_spec],
    out_shape=[
        jax.ShapeDtypeStruct((NPAD, F), jnp.float32),
        jax.ShapeDtypeStruct((NPAD, F), jnp.float32),
    ],
)

_mid_call = pl.pallas_call(
    _mid_body,
    grid=(),
    in_specs=[
        _node_spec,
        _part_spec,
        _node_spec,
        pl.BlockSpec((1, F), lambda: (0, 0)),
        pl.BlockSpec((F, F), lambda: (0, 0)),
    ],
    out_specs=_node_spec,
    out_shape=jax.ShapeDtypeStruct((NPAD, F), jnp.float32),
)

_post_call = pl.pallas_call(
    _post_body,
    grid=(),
    in_specs=[
        _node_spec,
        _part_spec,
        _node_spec,
        pl.BlockSpec((1, F), lambda: (0, 0)),
    ],
    out_specs=pl.BlockSpec((N_NODES, D_OUT), lambda: (0, 0)),
    out_shape=jax.ShapeDtypeStruct((N_NODES, D_OUT), jnp.float32),
)


def kernel(x, edge_index, W1, b1, W2, b2):
    ei = edge_index.astype(jnp.int32)
    pad = jnp.full((EPAD - N_EDGES,), N_NODES, jnp.int32)
    src_r = jnp.concatenate([ei[0], pad]).reshape(NW, NCH, CHUNK)
    dst_r = jnp.concatenate([ei[1], pad]).reshape(NW, NCH, CHUNK)
    xp = jnp.concatenate(
        [x, jnp.zeros((NPAD - N_NODES, D_IN), jnp.float32)], axis=0)
    w2p = jnp.concatenate(
        [W2, jnp.zeros((D_HID, F - D_OUT), jnp.float32)], axis=1)
    b1r = b1.reshape(1, D_HID)
    b2r = jnp.concatenate(
        [b2, jnp.zeros((F - D_OUT,), jnp.float32)]).reshape(1, F)

    degp = _sc_degree(dst_r)
    g1, dis = _pre_call(xp, W1, degp)
    agg1 = _sc_agg(g1, src_r, dst_r)
    g2 = _mid_call(g1, agg1, dis, b1r, w2p)
    agg2 = _sc_agg(g2, src_r, dst_r)
    return _post_call(g2, agg2, dis, b2r)


# speedup vs baseline: 52.1750x; 1.0368x over previous
import functools

import jax
import jax.numpy as jnp
from jax import lax
from jax.experimental import pallas as pl
from jax.experimental.pallas import tpu as pltpu
from jax.experimental.pallas import tpu_sc as plsc

N_NODES = 10000
N_EDGES = 320000
D_IN = 128
D_HID = 16
D_OUT = 12

NC, NS = 2, 16
NW = NC * NS
CHUNK = 128
NCH = 80
EPW = NCH * CHUNK
EPAD = EPW * NW
NPAD = 10240
RPS = NPAD // NS
F = 16
BLK = 512
NBLK = NPAD // BLK

_MESH = plsc.VectorSubcoreMesh(core_axis_name="c", subcore_axis_name="s")
_SC_PARAMS = pltpu.CompilerParams(use_tc_tiling_on_sc=False)


@functools.partial(
    pl.kernel,
    out_type=jax.ShapeDtypeStruct((NC, NPAD, F), jnp.float32),
    mesh=_MESH,
    scratch_types=[
        pltpu.VMEM((CHUNK, F), jnp.float32),
        pltpu.VMEM((NCH, CHUNK), jnp.int32),
        pltpu.VMEM_SHARED((NPAD, F), jnp.float32),
        pltpu.SemaphoreType.DMA,
        pltpu.SemaphoreType.DMA,
    ],
    compiler_params=_SC_PARAMS,
)
def _sc_degree(zeros_hbm, ones_hbm, dst_hbm, out_hbm, ones, idx, deg_sp,
               ssem, isem):
    cid = lax.axis_index("c")
    sid = lax.axis_index("s")
    w = cid * NS + sid

    d0 = pltpu.async_copy(zeros_hbm.at[pl.ds(sid * RPS, RPS)],
                          deg_sp.at[pl.ds(sid * RPS, RPS)], isem)
    d1 = pltpu.async_copy(ones_hbm, ones, isem)
    d2 = pltpu.async_copy(dst_hbm.at[w], idx, isem)
    d0.wait()
    d1.wait()
    d2.wait()
    plsc.subcore_barrier()

    def body(t, c):
        base = t * 16
        ds = [pltpu.async_copy(ones, deg_sp.at[idx.at[base + b]], ssem,
                               add=True)
              for b in range(16)]
        for d in ds:
            d.wait()
        return c
    lax.fori_loop(0, NCH // 16, body, 0)

    plsc.subcore_barrier()
    pltpu.sync_copy(deg_sp.at[pl.ds(sid * RPS, RPS)],
                    out_hbm.at[cid, pl.ds(sid * RPS, RPS)])


@functools.partial(
    pl.kernel,
    out_type=jax.ShapeDtypeStruct((NC, NPAD, F), jnp.float32),
    mesh=_MESH,
    scratch_types=[
        pltpu.VMEM((NCH, CHUNK), jnp.int32),
        pltpu.VMEM((NCH, CHUNK), jnp.int32),
        pltpu.VMEM((8, CHUNK, F), jnp.float32),
        pltpu.VMEM_SHARED((NPAD, F), jnp.float32),
        pltpu.VMEM_SHARED((NPAD, F), jnp.float32),
        pltpu.SemaphoreType.DMA,
        pltpu.SemaphoreType.DMA,
    ],
    compiler_params=_SC_PARAMS,
)
def _sc_agg(zeros_hbm, g_hbm, src_hbm, dst_hbm, out_hbm, sidx, didx, rows,
            agg_sp, g_sp, gsem, ssem):
    cid = lax.axis_index("c")
    sid = lax.axis_index("s")
    w = cid * NS + sid

    d0 = pltpu.async_copy(zeros_hbm.at[pl.ds(sid * RPS, RPS)],
                          agg_sp.at[pl.ds(sid * RPS, RPS)], gsem)
    d1 = pltpu.async_copy(g_hbm.at[pl.ds(sid * RPS, RPS)],
                          g_sp.at[pl.ds(sid * RPS, RPS)], gsem)
    d2 = pltpu.async_copy(src_hbm.at[w], sidx, gsem)
    d3 = pltpu.async_copy(dst_hbm.at[w], didx, gsem)
    d0.wait()
    d1.wait()
    d2.wait()
    d3.wait()
    plsc.subcore_barrier()

    def body(t, c):
        base = t * 8
        gds = [pltpu.async_copy(g_sp.at[sidx.at[base + b]], rows.at[b], gsem)
               for b in range(8)]
        for d in gds:
            d.wait()
        sds = [pltpu.async_copy(rows.at[b], agg_sp.at[didx.at[base + b]],
                                ssem, add=True)
               for b in range(8)]
        for d in sds:
            d.wait()
        return c
    lax.fori_loop(0, NCH // 8, body, 0)

    plsc.subcore_barrier()
    pltpu.sync_copy(agg_sp.at[pl.ds(sid * RPS, RPS)],
                    out_hbm.at[cid, pl.ds(sid * RPS, RPS)])


def _mm_body(x_ref, w1_ref, h1_ref):
    h1_ref[...] = jnp.dot(x_ref[...], w1_ref[...],
                          preferred_element_type=jnp.float32)


def _scale_body(h1_ref, degp_ref, g1_ref, dis_ref):
    deg = degp_ref[0] + degp_ref[1] + 1.0
    dis = lax.rsqrt(deg)
    g1_ref[...] = dis * h1_ref[...]
    dis_ref[...] = dis


def _mid_body(g1_ref, aggp_ref, dis_ref, b1_ref, w2_ref, g2_ref):
    dis = dis_ref[...]
    o = dis * (aggp_ref[0] + aggp_ref[1] + g1_ref[...]) + b1_ref[...]
    o = jnp.maximum(o, 0.0)
    h2 = jnp.dot(o, w2_ref[...], preferred_element_type=jnp.float32)
    g2_ref[...] = dis * h2


def _post_body(g2_ref, aggp_ref, dis_ref, b2_ref, out_ref):
    o = dis_ref[...] * (aggp_ref[0] + aggp_ref[1] + g2_ref[...]) + b2_ref[...]
    col = lax.broadcasted_iota(jnp.int32, (NPAD, F), 1)
    valid = col < D_OUT
    mx = jnp.max(jnp.where(valid, o, -jnp.inf), axis=1, keepdims=True)
    ex = jnp.where(valid, jnp.exp(o - mx), 0.0)
    lse = jnp.log(jnp.sum(ex, axis=1, keepdims=True))
    out_ref[...] = (o - mx - lse)[:N_NODES, :D_OUT]


_node_spec = pl.BlockSpec((NPAD, F), lambda: (0, 0))
_part_spec = pl.BlockSpec((NC, NPAD, F), lambda: (0, 0, 0))

_mm_call = pl.pallas_call(
    _mm_body,
    grid=(),
    in_specs=[
        pl.BlockSpec((NPAD, D_IN), lambda: (0, 0)),
        pl.BlockSpec((D_IN, D_HID), lambda: (0, 0)),
    ],
    out_specs=_node_spec,
    out_shape=jax.ShapeDtypeStruct((NPAD, F), jnp.float32),
)

_scale_call = pl.pallas_call(
    _scale_body,
    grid=(),
    in_specs=[_node_spec, _part_spec],
    out_specs=[_node_spec, _node_spec],
    out_shape=[
        jax.ShapeDtypeStruct((NPAD, F), jnp.float32),
        jax.ShapeDtypeStruct((NPAD, F), jnp.float32),
    ],
)

_mid_call = pl.pallas_call(
    _mid_body,
    grid=(),
    in_specs=[
        _node_spec,
        _part_spec,
        _node_spec,
        pl.BlockSpec((1, F), lambda: (0, 0)),
        pl.BlockSpec((F, F), lambda: (0, 0)),
    ],
    out_specs=_node_spec,
    out_shape=jax.ShapeDtypeStruct((NPAD, F), jnp.float32),
)

_post_call = pl.pallas_call(
    _post_body,
    grid=(),
    in_specs=[
        _node_spec,
        _part_spec,
        _node_spec,
        pl.BlockSpec((1, F), lambda: (0, 0)),
    ],
    out_specs=pl.BlockSpec((N_NODES, D_OUT), lambda: (0, 0)),
    out_shape=jax.ShapeDtypeStruct((N_NODES, D_OUT), jnp.float32),
)


def kernel(x, edge_index, W1, b1, W2, b2):
    ei = edge_index.astype(jnp.int32)
    pad = jnp.full((EPAD - N_EDGES,), N_NODES, jnp.int32)
    src_r = jnp.concatenate([ei[0], pad]).reshape(NW, NCH, CHUNK)
    dst_r = jnp.concatenate([ei[1], pad]).reshape(NW, NCH, CHUNK)
    xp = jnp.concatenate(
        [x, jnp.zeros((NPAD - N_NODES, D_IN), jnp.float32)], axis=0)
    w2p = jnp.concatenate(
        [W2, jnp.zeros((D_HID, F - D_OUT), jnp.float32)], axis=1)
    b1r = b1.reshape(1, D_HID)
    b2r = jnp.concatenate(
        [b2, jnp.zeros((F - D_OUT,), jnp.float32)]).reshape(1, F)
    znodes = jnp.zeros((NPAD, F), jnp.float32)
    orows = jnp.ones((CHUNK, F), jnp.float32)

    degp = _sc_degree(znodes, orows, dst_r)
    h1 = _mm_call(xp, W1)
    g1, dis = _scale_call(h1, degp)
    agg1 = _sc_agg(znodes, g1, src_r, dst_r)
    g2 = _mid_call(g1, agg1, dis, b1r, w2p)
    agg2 = _sc_agg(znodes, g2, src_r, dst_r)
    return _post_call(g2, agg2, dis, b2r)


# speedup vs baseline: 66.4160x; 1.2729x over previous
import functools

import jax
import jax.numpy as jnp
from jax import lax
from jax.experimental import pallas as pl
from jax.experimental.pallas import tpu as pltpu
from jax.experimental.pallas import tpu_sc as plsc

N_NODES = 10000
N_EDGES = 320000
D_IN = 128
D_HID = 16
D_OUT = 12

NC, NS = 2, 16
NW = NC * NS
CHUNK = 128
NCH = 80
EPW = NCH * CHUNK
EPAD = EPW * NW
NPAD = 10240
RPS = NPAD // NS
F = 16
BLK = 512
NBLK = NPAD // BLK

_MESH = plsc.VectorSubcoreMesh(core_axis_name="c", subcore_axis_name="s")
_SC_PARAMS = pltpu.CompilerParams(use_tc_tiling_on_sc=False)


@functools.partial(
    pl.kernel,
    out_type=jax.ShapeDtypeStruct((NC, NPAD, F), jnp.float32),
    mesh=_MESH,
    scratch_types=[
        pltpu.VMEM((CHUNK, F), jnp.float32),
        pltpu.VMEM((NCH, CHUNK), jnp.int32),
        pltpu.VMEM_SHARED((NPAD, F), jnp.float32),
        pltpu.SemaphoreType.DMA,
        pltpu.SemaphoreType.DMA,
    ],
    compiler_params=_SC_PARAMS,
)
def _sc_degree(zeros_hbm, ones_hbm, dst_hbm, out_hbm, ones, idx, deg_sp,
               ssem, isem):
    cid = lax.axis_index("c")
    sid = lax.axis_index("s")
    w = cid * NS + sid

    d0 = pltpu.async_copy(zeros_hbm.at[pl.ds(sid * RPS, RPS)],
                          deg_sp.at[pl.ds(sid * RPS, RPS)], isem)
    d1 = pltpu.async_copy(ones_hbm, ones, isem)
    d2 = pltpu.async_copy(dst_hbm.at[w], idx, isem)
    d0.wait()
    d1.wait()
    d2.wait()
    plsc.subcore_barrier()

    def body(t, c):
        base = t * 16
        ds = [pltpu.async_copy(ones, deg_sp.at[idx.at[base + b]], ssem,
                               add=True)
              for b in range(16)]
        for d in ds:
            d.wait()
        return c
    lax.fori_loop(0, NCH // 16, body, 0)

    plsc.subcore_barrier()
    pltpu.sync_copy(deg_sp.at[pl.ds(sid * RPS, RPS)],
                    out_hbm.at[cid, pl.ds(sid * RPS, RPS)])


@functools.partial(
    pl.kernel,
    out_type=jax.ShapeDtypeStruct((NC, NPAD, F), jnp.float32),
    mesh=_MESH,
    scratch_types=[
        pltpu.VMEM((NCH, CHUNK), jnp.int32),
        pltpu.VMEM((NCH, CHUNK), jnp.int32),
        pltpu.VMEM((8, CHUNK, F), jnp.float32),
        pltpu.VMEM_SHARED((NPAD, F), jnp.float32),
        pltpu.VMEM_SHARED((NPAD, F), jnp.float32),
        pltpu.SemaphoreType.DMA,
        pltpu.SemaphoreType.DMA,
    ],
    compiler_params=_SC_PARAMS,
)
def _sc_agg(zeros_hbm, g_hbm, src_hbm, dst_hbm, out_hbm, sidx, didx, rows,
            agg_sp, g_sp, gsem, ssem):
    cid = lax.axis_index("c")
    sid = lax.axis_index("s")
    w = cid * NS + sid

    d0 = pltpu.async_copy(zeros_hbm.at[pl.ds(sid * RPS, RPS)],
                          agg_sp.at[pl.ds(sid * RPS, RPS)], gsem)
    d1 = pltpu.async_copy(g_hbm.at[pl.ds(sid * RPS, RPS)],
                          g_sp.at[pl.ds(sid * RPS, RPS)], gsem)
    d2 = pltpu.async_copy(src_hbm.at[w], sidx, gsem)
    d3 = pltpu.async_copy(dst_hbm.at[w], didx, gsem)
    d0.wait()
    d1.wait()
    d2.wait()
    d3.wait()
    plsc.subcore_barrier()

    def body(t, c):
        base = t * 8
        gds = [pltpu.async_copy(g_sp.at[sidx.at[base + b]], rows.at[b], gsem)
               for b in range(8)]
        for d in gds:
            d.wait()
        sds = [pltpu.async_copy(rows.at[b], agg_sp.at[didx.at[base + b]],
                                ssem, add=True)
               for b in range(8)]
        for d in sds:
            d.wait()
        return c
    lax.fori_loop(0, NCH // 8, body, 0)

    plsc.subcore_barrier()
    pltpu.sync_copy(agg_sp.at[pl.ds(sid * RPS, RPS)],
                    out_hbm.at[cid, pl.ds(sid * RPS, RPS)])


RP = NPAD // 8
PW = 8 * F


def _mm_body(x_ref, w1_ref, h1_ref):
    h1_ref[...] = jnp.dot(x_ref[...], w1_ref[...],
                          preferred_element_type=jnp.float32)


def _scale_body(h1_ref, degp_ref, g1_ref, dis_ref):
    deg = degp_ref[0] + degp_ref[1] + 1.0
    dis = lax.rsqrt(deg)
    g1_ref[...] = dis * h1_ref[...]
    dis_ref[...] = dis


def _mid_body(g1_ref, aggp_ref, dis_ref, b1_ref, w2_ref, g2_ref):
    dis = dis_ref[...]
    o = dis * (aggp_ref[0] + aggp_ref[1] + g1_ref[...]) + b1_ref[...]
    o = jnp.maximum(o, 0.0)
    h2 = jnp.dot(o, w2_ref[...], preferred_element_type=jnp.float32)
    g2_ref[...] = dis * h2


def _post_body(g2_ref, aggp_ref, dis_ref, b2_ref, out_ref):
    o = dis_ref[...] * (aggp_ref[0] + aggp_ref[1] + g2_ref[...]) + b2_ref[...]
    col = lax.broadcasted_iota(jnp.int32, (NPAD, F), 1)
    valid = col < D_OUT
    mx = jnp.max(jnp.where(valid, o, -jnp.inf), axis=1, keepdims=True)
    ex = jnp.where(valid, jnp.exp(o - mx), 0.0)
    lse = jnp.log(jnp.sum(ex, axis=1, keepdims=True))
    out_ref[...] = (o - mx - lse)[:N_NODES, :D_OUT]


_pk_spec = pl.BlockSpec((RP, PW), lambda: (0, 0))
_pkpart_spec = pl.BlockSpec((NC, RP, PW), lambda: (0, 0, 0))
_pk_shape = jax.ShapeDtypeStruct((RP, PW), jnp.float32)

_mm_call = pl.pallas_call(
    _mm_body,
    grid=(),
    in_specs=[
        pl.BlockSpec((RP, 8 * D_IN), lambda: (0, 0)),
        pl.BlockSpec((8 * D_IN, PW), lambda: (0, 0)),
    ],
    out_specs=_pk_spec,
    out_shape=_pk_shape,
)

_scale_call = pl.pallas_call(
    _scale_body,
    grid=(),
    in_specs=[_pk_spec, _pkpart_spec],
    out_specs=[_pk_spec, _pk_spec],
    out_shape=[_pk_shape, _pk_shape],
)

_mid_call = pl.pallas_call(
    _mid_body,
    grid=(),
    in_specs=[
        _pk_spec,
        _pkpart_spec,
        _pk_spec,
        pl.BlockSpec((1, PW), lambda: (0, 0)),
        pl.BlockSpec((PW, PW), lambda: (0, 0)),
    ],
    out_specs=_pk_spec,
    out_shape=_pk_shape,
)

_post_call = pl.pallas_call(
    _post_body,
    grid=(),
    in_specs=[
        pl.BlockSpec((NPAD, F), lambda: (0, 0)),
        pl.BlockSpec((NC, NPAD, F), lambda: (0, 0, 0)),
        pl.BlockSpec((NPAD, F), lambda: (0, 0)),
        pl.BlockSpec((1, F), lambda: (0, 0)),
    ],
    out_specs=pl.BlockSpec((N_NODES, D_OUT), lambda: (0, 0)),
    out_shape=jax.ShapeDtypeStruct((N_NODES, D_OUT), jnp.float32),
)


def kernel(x, edge_index, W1, b1, W2, b2):
    ei = edge_index.astype(jnp.int32)
    pad = jnp.full((EPAD - N_EDGES,), N_NODES, jnp.int32)
    src_r = jnp.concatenate([ei[0], pad]).reshape(NW, NCH, CHUNK)
    dst_r = jnp.concatenate([ei[1], pad]).reshape(NW, NCH, CHUNK)
    xp = jnp.concatenate(
        [x, jnp.zeros((NPAD - N_NODES, D_IN), jnp.float32)],
        axis=0).reshape(RP, 8 * D_IN)
    w2p = jnp.concatenate(
        [W2, jnp.zeros((D_HID, F - D_OUT), jnp.float32)], axis=1)
    eye8 = jnp.eye(8, dtype=jnp.float32)
    w1blk = (eye8[:, None, :, None] * W1[None, :, None, :]).reshape(
        8 * D_IN, PW)
    w2blk = (eye8[:, None, :, None] * w2p[None, :, None, :]).reshape(PW, PW)
    b1r = jnp.tile(jnp.concatenate(
        [b1, jnp.zeros((F - D_HID,), jnp.float32)]), 8).reshape(1, PW)
    b2r = jnp.concatenate(
        [b2, jnp.zeros((F - D_OUT,), jnp.float32)]).reshape(1, F)
    znodes = jnp.zeros((NPAD, F), jnp.float32)
    orows = jnp.ones((CHUNK, F), jnp.float32)

    degp = _sc_degree(znodes, orows, dst_r)
    h1 = _mm_call(xp, w1blk)
    g1, dis = _scale_call(h1, degp.reshape(NC, RP, PW))
    agg1 = _sc_agg(znodes, g1.reshape(NPAD, F), src_r, dst_r)
    g2 = _mid_call(g1, agg1.reshape(NC, RP, PW), dis, b1r, w2blk)
    g2u = g2.reshape(NPAD, F)
    agg2 = _sc_agg(znodes, g2u, src_r, dst_r)
    return _post_call(g2u, agg2, dis.reshape(NPAD, F), b2r)


# speedup vs baseline: 67.4754x; 1.0160x over previous
import functools

import jax
import jax.numpy as jnp
from jax import lax
from jax.experimental import pallas as pl
from jax.experimental.pallas import tpu as pltpu
from jax.experimental.pallas import tpu_sc as plsc

N_NODES = 10000
N_EDGES = 320000
D_IN = 128
D_HID = 16
D_OUT = 12

NC, NS = 2, 16
NW = NC * NS
CHUNK = 128
NCH = 80
EPW = NCH * CHUNK
EPAD = EPW * NW
NPAD = 10240
RPS = NPAD // NS
F = 16
BLK = 512
NBLK = NPAD // BLK

_MESH = plsc.VectorSubcoreMesh(core_axis_name="c", subcore_axis_name="s")
_SC_PARAMS = pltpu.CompilerParams(use_tc_tiling_on_sc=False)


@functools.partial(
    pl.kernel,
    out_type=jax.ShapeDtypeStruct((NC, NPAD, F), jnp.float32),
    mesh=_MESH,
    scratch_types=[
        pltpu.VMEM((CHUNK, F), jnp.float32),
        pltpu.VMEM((NCH, CHUNK), jnp.int32),
        pltpu.VMEM_SHARED((NPAD, F), jnp.float32),
        pltpu.SemaphoreType.DMA,
        pltpu.SemaphoreType.DMA,
    ],
    compiler_params=_SC_PARAMS,
)
def _sc_degree(zeros_hbm, ones_hbm, dst_hbm, out_hbm, ones, idx, deg_sp,
               ssem, isem):
    cid = lax.axis_index("c")
    sid = lax.axis_index("s")
    w = cid * NS + sid

    d0 = pltpu.async_copy(zeros_hbm.at[pl.ds(sid * RPS, RPS)],
                          deg_sp.at[pl.ds(sid * RPS, RPS)], isem)
    d1 = pltpu.async_copy(ones_hbm, ones, isem)
    d2 = pltpu.async_copy(dst_hbm.at[w], idx, isem)
    d0.wait()
    d1.wait()
    d2.wait()
    plsc.subcore_barrier()

    def body(t, c):
        base = t * 16
        ds = [pltpu.async_copy(ones, deg_sp.at[idx.at[base + b]], ssem,
                               add=True)
              for b in range(16)]
        for d in ds:
            d.wait()
        return c
    lax.fori_loop(0, NCH // 16, body, 0)

    plsc.subcore_barrier()
    pltpu.sync_copy(deg_sp.at[pl.ds(sid * RPS, RPS)],
                    out_hbm.at[cid, pl.ds(sid * RPS, RPS)])


@functools.partial(
    pl.kernel,
    out_type=jax.ShapeDtypeStruct((NC, NPAD, F), jnp.float32),
    mesh=_MESH,
    scratch_types=[
        pltpu.VMEM((NCH, CHUNK), jnp.int32),
        pltpu.VMEM((NCH, CHUNK), jnp.int32),
        pltpu.VMEM((8, CHUNK, F), jnp.float32),
        pltpu.VMEM_SHARED((NPAD, F), jnp.float32),
        pltpu.VMEM_SHARED((NPAD, F), jnp.float32),
        pltpu.SemaphoreType.DMA,
        pltpu.SemaphoreType.DMA,
    ],
    compiler_params=_SC_PARAMS,
)
def _sc_agg(zeros_hbm, g_hbm, src_hbm, dst_hbm, out_hbm, sidx, didx, rows,
            agg_sp, g_sp, gsem, ssem):
    cid = lax.axis_index("c")
    sid = lax.axis_index("s")
    w = cid * NS + sid

    d0 = pltpu.async_copy(zeros_hbm.at[pl.ds(sid * RPS, RPS)],
                          agg_sp.at[pl.ds(sid * RPS, RPS)], gsem)
    d1 = pltpu.async_copy(g_hbm.at[pl.ds(sid * RPS, RPS)],
                          g_sp.at[pl.ds(sid * RPS, RPS)], gsem)
    d2 = pltpu.async_copy(src_hbm.at[w], sidx, gsem)
    d3 = pltpu.async_copy(dst_hbm.at[w], didx, gsem)
    d0.wait()
    d1.wait()
    d2.wait()
    d3.wait()
    plsc.subcore_barrier()

    pend = [None, None]
    for t in range(NCH // 4):
        half = t % 2
        if pend[half] is not None:
            for d in pend[half]:
                d.wait()
        base = t * 4
        gds = [pltpu.async_copy(g_sp.at[sidx.at[base + b]],
                                rows.at[half * 4 + b], gsem)
               for b in range(4)]
        for d in gds:
            d.wait()
        pend[half] = [pltpu.async_copy(rows.at[half * 4 + b],
                                       agg_sp.at[didx.at[base + b]],
                                       ssem, add=True)
                      for b in range(4)]
    for p in pend:
        for d in p:
            d.wait()

    plsc.subcore_barrier()
    pltpu.sync_copy(agg_sp.at[pl.ds(sid * RPS, RPS)],
                    out_hbm.at[cid, pl.ds(sid * RPS, RPS)])


RP = NPAD // 8
PW = 8 * F


def _mm_body(x_ref, w1_ref, h1_ref):
    h1_ref[...] = jnp.dot(x_ref[...], w1_ref[...],
                          preferred_element_type=jnp.float32)


def _scale_body(h1_ref, degp_ref, g1_ref, dis_ref):
    deg = degp_ref[0] + degp_ref[1] + 1.0
    dis = lax.rsqrt(deg)
    g1_ref[...] = dis * h1_ref[...]
    dis_ref[...] = dis


def _mid_body(g1_ref, aggp_ref, dis_ref, b1_ref, w2_ref, g2_ref):
    dis = dis_ref[...]
    o = dis * (aggp_ref[0] + aggp_ref[1] + g1_ref[...]) + b1_ref[...]
    o = jnp.maximum(o, 0.0)
    h2 = jnp.dot(o, w2_ref[...], preferred_element_type=jnp.float32)
    g2_ref[...] = dis * h2


def _post_body(g2_ref, aggp_ref, dis_ref, b2_ref, out_ref):
    o = dis_ref[...] * (aggp_ref[0] + aggp_ref[1] + g2_ref[...]) + b2_ref[...]
    col = lax.broadcasted_iota(jnp.int32, (NPAD, F), 1)
    valid = col < D_OUT
    mx = jnp.max(jnp.where(valid, o, -jnp.inf), axis=1, keepdims=True)
    ex = jnp.where(valid, jnp.exp(o - mx), 0.0)
    lse = jnp.log(jnp.sum(ex, axis=1, keepdims=True))
    out_ref[...] = (o - mx - lse)[:N_NODES, :D_OUT]


_pk_spec = pl.BlockSpec((RP, PW), lambda: (0, 0))
_pkpart_spec = pl.BlockSpec((NC, RP, PW), lambda: (0, 0, 0))
_pk_shape = jax.ShapeDtypeStruct((RP, PW), jnp.float32)

_mm_call = pl.pallas_call(
    _mm_body,
    grid=(),
    in_specs=[
        pl.BlockSpec((RP, 8 * D_IN), lambda: (0, 0)),
        pl.BlockSpec((8 * D_IN, PW), lambda: (0, 0)),
    ],
    out_specs=_pk_spec,
    out_shape=_pk_shape,
)

_scale_call = pl.pallas_call(
    _scale_body,
    grid=(),
    in_specs=[_pk_spec, _pkpart_spec],
    out_specs=[_pk_spec, _pk_spec],
    out_shape=[_pk_shape, _pk_shape],
)

_mid_call = pl.pallas_call(
    _mid_body,
    grid=(),
    in_specs=[
        _pk_spec,
        _pkpart_spec,
        _pk_spec,
        pl.BlockSpec((1, PW), lambda: (0, 0)),
        pl.BlockSpec((PW, PW), lambda: (0, 0)),
    ],
    out_specs=_pk_spec,
    out_shape=_pk_shape,
)

_post_call = pl.pallas_call(
    _post_body,
    grid=(),
    in_specs=[
        pl.BlockSpec((NPAD, F), lambda: (0, 0)),
        pl.BlockSpec((NC, NPAD, F), lambda: (0, 0, 0)),
        pl.BlockSpec((NPAD, F), lambda: (0, 0)),
        pl.BlockSpec((1, F), lambda: (0, 0)),
    ],
    out_specs=pl.BlockSpec((N_NODES, D_OUT), lambda: (0, 0)),
    out_shape=jax.ShapeDtypeStruct((N_NODES, D_OUT), jnp.float32),
)


def kernel(x, edge_index, W1, b1, W2, b2):
    ei = edge_index.astype(jnp.int32)
    pad = jnp.full((EPAD - N_EDGES,), N_NODES, jnp.int32)
    src_r = jnp.concatenate([ei[0], pad]).reshape(NW, NCH, CHUNK)
    dst_r = jnp.concatenate([ei[1], pad]).reshape(NW, NCH, CHUNK)
    xp = jnp.concatenate(
        [x, jnp.zeros((NPAD - N_NODES, D_IN), jnp.float32)],
        axis=0).reshape(RP, 8 * D_IN)
    w2p = jnp.concatenate(
        [W2, jnp.zeros((D_HID, F - D_OUT), jnp.float32)], axis=1)
    eye8 = jnp.eye(8, dtype=jnp.float32)
    w1blk = (eye8[:, None, :, None] * W1[None, :, None, :]).reshape(
        8 * D_IN, PW)
    w2blk = (eye8[:, None, :, None] * w2p[None, :, None, :]).reshape(PW, PW)
    b1r = jnp.tile(jnp.concatenate(
        [b1, jnp.zeros((F - D_HID,), jnp.float32)]), 8).reshape(1, PW)
    b2r = jnp.concatenate(
        [b2, jnp.zeros((F - D_OUT,), jnp.float32)]).reshape(1, F)
    znodes = jnp.zeros((NPAD, F), jnp.float32)
    orows = jnp.ones((CHUNK, F), jnp.float32)

    degp = _sc_degree(znodes, orows, dst_r)
    h1 = _mm_call(xp, w1blk)
    g1, dis = _scale_call(h1, degp.reshape(NC, RP, PW))
    agg1 = _sc_agg(znodes, g1.reshape(NPAD, F), src_r, dst_r)
    g2 = _mid_call(g1, agg1.reshape(NC, RP, PW), dis, b1r, w2blk)
    g2u = g2.reshape(NPAD, F)
    agg2 = _sc_agg(znodes, g2u, src_r, dst_r)
    return _post_call(g2u, agg2, dis.reshape(NPAD, F), b2r)
